# baseline (speedup 1.0000x reference)
import functools

import jax
import jax.numpy as jnp
from jax import lax
from jax.experimental import pallas as pl
from jax.experimental.pallas import tpu as pltpu
from jax.experimental.pallas import tpu_sc as plsc

_N = 10000
_DEG = 16
_E = _N * _DEG
_NEG = 0.01

_NW = 32
_CH = 1000
_TN = 2000


def _sc_gather(table, idx, D):
    E = idx.shape[0]
    per_w = E // _NW
    n_ch = per_w // _CH
    mesh = plsc.VectorSubcoreMesh(core_axis_name="c", subcore_axis_name="s")

    @functools.partial(
        pl.kernel,
        out_type=jax.ShapeDtypeStruct((E, D), jnp.float32),
        mesh=mesh,
        scratch_types=[
            pltpu.VMEM((_CH,), jnp.int32),
            pltpu.VMEM((_CH, D), jnp.float32),
            pltpu.SemaphoreType.DMA,
        ],
    )
    def gather_kernel(table_hbm, idx_hbm, out_hbm, idx_v, rows_v, sem):
        wid = lax.axis_index("s") * 2 + lax.axis_index("c")
        base = wid * per_w
        for i in range(n_ch):
            off = base + i * _CH
            pltpu.sync_copy(idx_hbm.at[pl.ds(off, _CH)], idx_v)
            pltpu.async_copy(table_hbm.at[idx_v], rows_v, sem).wait()
            pltpu.sync_copy(rows_v, out_hbm.at[pl.ds(off, _CH)])

    return gather_kernel(table, idx)


def _embed(x, xc, emb_cat):
    def body(x_ref, xc_ref, e_ref, o_ref):
        xv = x_ref[...].astype(jnp.int32)
        oh0 = (xv[:, 0:1] == lax.broadcasted_iota(jnp.int32, (_TN, 14), 1))
        oh1 = (xv[:, 1:2] == lax.broadcasted_iota(jnp.int32, (_TN, 5), 1))
        oh2 = (xv[:, 2:3] == lax.broadcasted_iota(jnp.int32, (_TN, 10), 1))
        oh = jnp.concatenate(
            [oh0.astype(jnp.float32), oh1.astype(jnp.float32),
             oh2.astype(jnp.float32)], axis=1)
        e = jnp.dot(oh, e_ref[...], preferred_element_type=jnp.float32)
        o_ref[...] = jnp.concatenate(
            [e, xc_ref[...], jnp.zeros((_TN, 82), jnp.float32)], axis=1)

    return pl.pallas_call(
        body,
        grid=(_N // _TN,),
        in_specs=[
            pl.BlockSpec((_TN, 3), lambda i: (i, 0)),
            pl.BlockSpec((_TN, 34), lambda i: (i, 0)),
            pl.BlockSpec((29, 12), lambda i: (0, 0)),
        ],
        out_specs=pl.BlockSpec((_TN, 128), lambda i: (i, 0)),
        out_shape=jax.ShapeDtypeStruct((_N, 128), jnp.float32),
    )(x, xc, emb_cat)


def _sage_layer(h, neigh, Wg, Ug, bg, Ws, bs, Wn, dp_in, dg, dout,
                extra=None):
    GP = 4 * dg
    d_s = 128 if extra is None else dout
    d_o = 10 if extra is not None else 128

    def body(h_ref, n_ref, Wg_ref, Ug_ref, bg_ref, Ws_ref, bs_ref, Wn_ref,
             *rest):
        o_ref = rest[-1]
        hv = h_ref[...]
        selfp = jnp.dot(hv, Ws_ref[...],
                        preferred_element_type=jnp.float32) + bs_ref[...]
        Wgv = Wg_ref[...]
        Ugv = Ug_ref[...]
        bgv = bg_ref[...]
        hs = jnp.zeros((_TN, dg), jnp.float32)
        c = jnp.zeros((_TN, dg), jnp.float32)
        for t in range(_DEG):
            gates = (jnp.dot(n_ref[t], Wgv, preferred_element_type=jnp.float32)
                     + jnp.dot(hs, Ugv, preferred_element_type=jnp.float32)
                     + bgv)
            s = jnp.tanh(gates)
            ti = s[:, 0:dg]
            tf = s[:, dg:2 * dg]
            tg = s[:, 2 * dg:3 * dg]
            to = s[:, 3 * dg:4 * dg]
            c = (0.5 + 0.5 * tf) * c + (0.5 + 0.5 * ti) * tg
            hs = (0.5 + 0.5 * to) * jnp.tanh(c)
        out = selfp + jnp.dot(hs, Wn_ref[...],
                              preferred_element_type=jnp.float32)
        out = jnp.where(out > 0, out, _NEG * out)
        if extra is not None:
            W1_ref, b1_ref, W2_ref, b2_ref = rest[:4]
            z = jnp.dot(out, W1_ref[...],
                        preferred_element_type=jnp.float32) + b1_ref[...]
            z = jnp.where(z > 0, z, _NEG * z)
            out = jnp.dot(z, W2_ref[...],
                          preferred_element_type=jnp.float32) + b2_ref[...]
        o_ref[...] = out

    in_specs = [
        pl.BlockSpec((_TN, dp_in), lambda i: (i, 0)),
        pl.BlockSpec((_DEG, _TN, dp_in), lambda i: (0, i, 0)),
        pl.BlockSpec((dp_in, GP), lambda i: (0, 0)),
        pl.BlockSpec((dg, GP), lambda i: (0, 0)),
        pl.BlockSpec((1, GP), lambda i: (0, 0)),
        pl.BlockSpec((dp_in, d_s), lambda i: (0, 0)),
        pl.BlockSpec((1, d_s), lambda i: (0, 0)),
        pl.BlockSpec((dg, d_s), lambda i: (0, 0)),
    ]
    args = [h, neigh, Wg, Ug, bg, Ws, bs, Wn]
    if extra is not None:
        W1, b1, W2, b2 = extra
        in_specs += [
            pl.BlockSpec((dout, 72), lambda i: (0, 0)),
            pl.BlockSpec((1, 72), lambda i: (0, 0)),
            pl.BlockSpec((72, 10), lambda i: (0, 0)),
            pl.BlockSpec((1, 10), lambda i: (0, 0)),
        ]
        args += [W1, b1, W2, b2]

    return pl.pallas_call(
        body,
        grid=(_N // _TN,),
        in_specs=in_specs,
        out_specs=pl.BlockSpec((_TN, d_o), lambda i: (i, 0)),
        out_shape=jax.ShapeDtypeStruct((_N, d_o), jnp.float32),
    )(*args)


def _pack_gates(Wih, Whh, bih, bhh, din, dp_in, dg):
    sc = jnp.array([0.5, 0.5, 1.0, 0.5], jnp.float32)
    Wg = jnp.concatenate(
        [sc[g] * jnp.pad(Wih[g * din:(g + 1) * din, :].T,
                         ((0, dp_in - din), (0, dg - din))) for g in range(4)],
        axis=1)
    Ug = jnp.concatenate(
        [sc[g] * jnp.pad(Whh[g * din:(g + 1) * din, :].T,
                         ((0, dg - din), (0, dg - din))) for g in range(4)],
        axis=1)
    b = bih + bhh
    bg = jnp.concatenate(
        [sc[g] * jnp.pad(b[g * din:(g + 1) * din], (0, dg - din))
         for g in range(4)])[None, :]
    return Wg, Ug, bg


def kernel(x, edge_index, params):
    src = edge_index[0].astype(jnp.int32)
    src_tm = src.reshape(_N, _DEG).T.reshape(-1)

    e0, e1, e2 = params['emb0'], params['emb1'], params['emb2']
    emb_cat = jnp.concatenate([
        jnp.pad(e0, ((0, 0), (0, 6))),
        jnp.pad(e1, ((0, 0), (6, 4))),
        jnp.pad(e2, ((0, 0), (8, 0))),
    ], axis=0)

    h = _embed(x[:, :3], x[:, 3:], emb_cat)

    dims = [(46, 128, 64, 64), (64, 128, 64, 80), (80, 128, 96, 96)]
    for l, (din, dp_in, dg, dout) in enumerate(dims):
        cpad = 0 if l == 2 else 128 - dout
        Wg, Ug, bg = _pack_gates(
            params['l%d_Wih' % l], params['l%d_Whh' % l],
            params['l%d_bih' % l], params['l%d_bhh' % l], din, dp_in, dg)
        Ws = jnp.pad(params['l%d_Wself' % l].T,
                     ((0, dp_in - din), (0, cpad)))
        bs = jnp.pad(params['l%d_bself' % l], (0, cpad))[None, :]
        Wn = jnp.pad(params['l%d_Wneigh' % l].T,
                     ((0, dg - din), (0, cpad)))
        extra = None
        if l == 2:
            extra = (params['lin1_W'].T, params['lin1_b'][None, :],
                     params['lin2_W'].T, params['lin2_b'][None, :])
        neigh = _sc_gather(h, src_tm, dp_in).reshape(_DEG, _N, dp_in)
        h = _sage_layer(h, neigh, Wg, Ug, bg, Ws, bs, Wn, dp_in, dg, dout,
                        extra=extra)
    return h


# speedup vs baseline: 1.0584x; 1.0584x over previous
import functools

import jax
import jax.numpy as jnp
from jax import lax
from jax.experimental import pallas as pl
from jax.experimental.pallas import tpu as pltpu
from jax.experimental.pallas import tpu_sc as plsc

_N = 10000
_DEG = 16
_E = _N * _DEG
_NEG = 0.01

_NW = 32
_CH = 1000
_TN = 1000


def _sc_gather(table, idx, D):
    E = idx.shape[0]
    per_w = E // _NW
    n_ch = per_w // _CH
    mesh = plsc.VectorSubcoreMesh(core_axis_name="c", subcore_axis_name="s")

    @functools.partial(
        pl.kernel,
        out_type=jax.ShapeDtypeStruct((E, D), jnp.float32),
        mesh=mesh,
        scratch_types=[
            pltpu.VMEM((_CH,), jnp.int32),
            pltpu.VMEM((_CH, D), jnp.float32),
            pltpu.SemaphoreType.DMA,
        ],
    )
    def gather_kernel(table_hbm, idx_hbm, out_hbm, idx_v, rows_v, sem):
        wid = lax.axis_index("s") * 2 + lax.axis_index("c")
        base = wid * per_w
        for i in range(n_ch):
            off = base + i * _CH
            pltpu.sync_copy(idx_hbm.at[pl.ds(off, _CH)], idx_v)
            pltpu.async_copy(table_hbm.at[idx_v], rows_v, sem).wait()
            pltpu.sync_copy(rows_v, out_hbm.at[pl.ds(off, _CH)])

    return gather_kernel(table, idx)


def _embed(x, xc, emb_cat):
    def body(x_ref, xc_ref, e_ref, o_ref):
        xv = x_ref[...].astype(jnp.int32)
        oh0 = (xv[:, 0:1] == lax.broadcasted_iota(jnp.int32, (_TN, 14), 1))
        oh1 = (xv[:, 1:2] == lax.broadcasted_iota(jnp.int32, (_TN, 5), 1))
        oh2 = (xv[:, 2:3] == lax.broadcasted_iota(jnp.int32, (_TN, 10), 1))
        oh = jnp.concatenate(
            [oh0.astype(jnp.float32), oh1.astype(jnp.float32),
             oh2.astype(jnp.float32)], axis=1)
        e = jnp.dot(oh, e_ref[...], preferred_element_type=jnp.float32)
        o_ref[...] = jnp.concatenate(
            [e, xc_ref[...], jnp.zeros((_TN, 82), jnp.float32)], axis=1)

    return pl.pallas_call(
        body,
        grid=(_N // _TN,),
        in_specs=[
            pl.BlockSpec((_TN, 3), lambda i: (i, 0)),
            pl.BlockSpec((_TN, 34), lambda i: (i, 0)),
            pl.BlockSpec((29, 12), lambda i: (0, 0)),
        ],
        out_specs=pl.BlockSpec((_TN, 128), lambda i: (i, 0)),
        out_shape=jax.ShapeDtypeStruct((_N, 128), jnp.float32),
    )(x, xc, emb_cat)


def _self_part(h, Ws, bs, dp_in, d_s):
    def body(h_ref, Ws_ref, bs_ref, o_ref):
        o_ref[...] = jnp.dot(h_ref[...], Ws_ref[...],
                             preferred_element_type=jnp.float32) + bs_ref[...]

    return pl.pallas_call(
        body,
        grid=(_N // _TN,),
        in_specs=[
            pl.BlockSpec((_TN, dp_in), lambda i: (i, 0)),
            pl.BlockSpec((dp_in, d_s), lambda i: (0, 0)),
            pl.BlockSpec((1, d_s), lambda i: (0, 0)),
        ],
        out_specs=pl.BlockSpec((_TN, d_s), lambda i: (i, 0)),
        out_shape=jax.ShapeDtypeStruct((_N, d_s), jnp.float32),
    )(h, Ws, bs)


def _sage_layer(selfp, neigh, Wg, Ug, bg, Wn, dp_in, dg, dout,
                extra=None):
    GP = 4 * dg
    d_s = 128 if extra is None else dout
    d_o = 10 if extra is not None else 128

    def body(sp_ref, n_ref, Wg_ref, Ug_ref, bg_ref, Wn_ref, *rest):
        o_ref = rest[-1]
        selfp = sp_ref[...]
        nv = n_ref[...].reshape(_DEG * _TN, dp_in)
        gin = jnp.dot(nv, Wg_ref[...],
                      preferred_element_type=jnp.float32) + bg_ref[...]
        Ugv = Ug_ref[...]
        hs = jnp.zeros((_TN, dg), jnp.float32)
        c = jnp.zeros((_TN, dg), jnp.float32)
        for t in range(_DEG):
            gates = (gin[t * _TN:(t + 1) * _TN, :]
                     + jnp.dot(hs, Ugv, preferred_element_type=jnp.float32))
            s = jnp.tanh(gates)
            ti = s[:, 0:dg]
            tf = s[:, dg:2 * dg]
            tg = s[:, 2 * dg:3 * dg]
            to = s[:, 3 * dg:4 * dg]
            c = (0.5 + 0.5 * tf) * c + (0.5 + 0.5 * ti) * tg
            hs = (0.5 + 0.5 * to) * jnp.tanh(c)
        out = selfp + jnp.dot(hs, Wn_ref[...],
                              preferred_element_type=jnp.float32)
        out = jnp.where(out > 0, out, _NEG * out)
        if extra is not None:
            W1_ref, b1_ref, W2_ref, b2_ref = rest[:4]
            z = jnp.dot(out, W1_ref[...],
                        preferred_element_type=jnp.float32) + b1_ref[...]
            z = jnp.where(z > 0, z, _NEG * z)
            out = jnp.dot(z, W2_ref[...],
                          preferred_element_type=jnp.float32) + b2_ref[...]
        o_ref[...] = out

    in_specs = [
        pl.BlockSpec((_TN, d_s), lambda i: (i, 0)),
        pl.BlockSpec((_DEG, _TN, dp_in), lambda i: (0, i, 0)),
        pl.BlockSpec((dp_in, GP), lambda i: (0, 0)),
        pl.BlockSpec((dg, GP), lambda i: (0, 0)),
        pl.BlockSpec((1, GP), lambda i: (0, 0)),
        pl.BlockSpec((dg, d_s), lambda i: (0, 0)),
    ]
    args = [selfp, neigh, Wg, Ug, bg, Wn]
    if extra is not None:
        W1, b1, W2, b2 = extra
        in_specs += [
            pl.BlockSpec((dout, 72), lambda i: (0, 0)),
            pl.BlockSpec((1, 72), lambda i: (0, 0)),
            pl.BlockSpec((72, 10), lambda i: (0, 0)),
            pl.BlockSpec((1, 10), lambda i: (0, 0)),
        ]
        args += [W1, b1, W2, b2]

    return pl.pallas_call(
        body,
        grid=(_N // _TN,),
        in_specs=in_specs,
        out_specs=pl.BlockSpec((_TN, d_o), lambda i: (i, 0)),
        out_shape=jax.ShapeDtypeStruct((_N, d_o), jnp.float32),
    )(*args)


def _pack_gates(Wih, Whh, bih, bhh, din, dp_in, dg):
    sc = jnp.array([0.5, 0.5, 1.0, 0.5], jnp.float32)
    Wg = jnp.concatenate(
        [sc[g] * jnp.pad(Wih[g * din:(g + 1) * din, :].T,
                         ((0, dp_in - din), (0, dg - din))) for g in range(4)],
        axis=1)
    Ug = jnp.concatenate(
        [sc[g] * jnp.pad(Whh[g * din:(g + 1) * din, :].T,
                         ((0, dg - din), (0, dg - din))) for g in range(4)],
        axis=1)
    b = bih + bhh
    bg = jnp.concatenate(
        [sc[g] * jnp.pad(b[g * din:(g + 1) * din], (0, dg - din))
         for g in range(4)])[None, :]
    return Wg, Ug, bg


def kernel(x, edge_index, params):
    src = edge_index[0].astype(jnp.int32)
    src_tm = src.reshape(_N, _DEG).T.reshape(-1)

    e0, e1, e2 = params['emb0'], params['emb1'], params['emb2']
    emb_cat = jnp.concatenate([
        jnp.pad(e0, ((0, 0), (0, 6))),
        jnp.pad(e1, ((0, 0), (6, 4))),
        jnp.pad(e2, ((0, 0), (8, 0))),
    ], axis=0)

    h = _embed(x[:, :3], x[:, 3:], emb_cat)

    dims = [(46, 128, 64, 64), (64, 128, 64, 80), (80, 128, 96, 96)]
    for l, (din, dp_in, dg, dout) in enumerate(dims):
        cpad = 0 if l == 2 else 128 - dout
        Wg, Ug, bg = _pack_gates(
            params['l%d_Wih' % l], params['l%d_Whh' % l],
            params['l%d_bih' % l], params['l%d_bhh' % l], din, dp_in, dg)
        Ws = jnp.pad(params['l%d_Wself' % l].T,
                     ((0, dp_in - din), (0, cpad)))
        bs = jnp.pad(params['l%d_bself' % l], (0, cpad))[None, :]
        Wn = jnp.pad(params['l%d_Wneigh' % l].T,
                     ((0, dg - din), (0, cpad)))
        extra = None
        if l == 2:
            extra = (params['lin1_W'].T, params['lin1_b'][None, :],
                     params['lin2_W'].T, params['lin2_b'][None, :])
        d_s = dout if l == 2 else 128
        selfp = _self_part(h, Ws, bs, dp_in, d_s)
        neigh = _sc_gather(h, src_tm, dp_in).reshape(_DEG, _N, dp_in)
        h = _sage_layer(selfp, neigh, Wg, Ug, bg, Wn, dp_in, dg, dout,
                        extra=extra)
    return h


# speedup vs baseline: 1.0648x; 1.0061x over previous
import functools

import jax
import jax.numpy as jnp
from jax import lax
from jax.experimental import pallas as pl
from jax.experimental.pallas import tpu as pltpu
from jax.experimental.pallas import tpu_sc as plsc

_N = 10000
_DEG = 16
_E = _N * _DEG
_NEG = 0.01

_NW = 32
_CH = 1000
_TN = 1000


def _sc_gather(table, idx, D):
    E = idx.shape[0]
    per_w = E // _NW
    n_ch = per_w // _CH
    mesh = plsc.VectorSubcoreMesh(core_axis_name="c", subcore_axis_name="s")

    @functools.partial(
        pl.kernel,
        out_type=jax.ShapeDtypeStruct((E, D), jnp.float32),
        mesh=mesh,
        scratch_types=[
            pltpu.VMEM((_CH,), jnp.int32),
            pltpu.VMEM((_CH, D), jnp.float32),
            pltpu.SemaphoreType.DMA,
            pltpu.SemaphoreType.DMA,
            pltpu.SemaphoreType.DMA,
        ],
    )
    def gather_kernel(table_hbm, idx_hbm, out_hbm, idx_v, rows_v,
                      gsem, wsem, isem):
        wid = lax.axis_index("s") * 2 + lax.axis_index("c")
        base = wid * per_w
        pltpu.sync_copy(idx_hbm.at[pl.ds(base, _CH)], idx_v)
        for i in range(n_ch):
            off = base + i * _CH
            pltpu.async_copy(table_hbm.at[idx_v], rows_v, gsem).wait()
            w = pltpu.async_copy(rows_v, out_hbm.at[pl.ds(off, _CH)], wsem)
            if i + 1 < n_ch:
                ic = pltpu.async_copy(
                    idx_hbm.at[pl.ds(off + _CH, _CH)], idx_v, isem)
                ic.wait()
            w.wait()

    return gather_kernel(table, idx)


def _embed(x, xc, emb_cat):
    def body(x_ref, xc_ref, e_ref, o_ref):
        xv = x_ref[...].astype(jnp.int32)
        oh0 = (xv[:, 0:1] == lax.broadcasted_iota(jnp.int32, (_TN, 14), 1))
        oh1 = (xv[:, 1:2] == lax.broadcasted_iota(jnp.int32, (_TN, 5), 1))
        oh2 = (xv[:, 2:3] == lax.broadcasted_iota(jnp.int32, (_TN, 10), 1))
        oh = jnp.concatenate(
            [oh0.astype(jnp.float32), oh1.astype(jnp.float32),
             oh2.astype(jnp.float32)], axis=1)
        e = jnp.dot(oh, e_ref[...], preferred_element_type=jnp.float32)
        o_ref[...] = jnp.concatenate(
            [e, xc_ref[...], jnp.zeros((_TN, 82), jnp.float32)], axis=1)

    return pl.pallas_call(
        body,
        grid=(_N // _TN,),
        in_specs=[
            pl.BlockSpec((_TN, 3), lambda i: (i, 0)),
            pl.BlockSpec((_TN, 34), lambda i: (i, 0)),
            pl.BlockSpec((29, 12), lambda i: (0, 0)),
        ],
        out_specs=pl.BlockSpec((_TN, 128), lambda i: (i, 0)),
        out_shape=jax.ShapeDtypeStruct((_N, 128), jnp.float32),
    )(x, xc, emb_cat)


def _sage_layer(h, neigh, Wg, Ug, bg, Ws, bs, Wn, dp_in, dg, dout,
                extra=None):
    GP = 4 * dg
    d_s = 128 if extra is None else dout
    d_o = 10 if extra is not None else 128

    def body(h_ref, n_ref, Wg_ref, Ug_ref, bg_ref, Ws_ref, bs_ref, Wn_ref,
             *rest):
        o_ref = rest[-1]
        hv = h_ref[...]
        selfp = jnp.dot(hv, Ws_ref[...],
                        preferred_element_type=jnp.float32) + bs_ref[...]
        nv = n_ref[...].reshape(_DEG * _TN, dp_in)
        gin = jnp.dot(nv, Wg_ref[...],
                      preferred_element_type=jnp.float32) + bg_ref[...]
        Ugv = Ug_ref[...]
        hs = jnp.zeros((_TN, dg), jnp.float32)
        c = jnp.zeros((_TN, dg), jnp.float32)
        for t in range(_DEG):
            gates = (gin[t * _TN:(t + 1) * _TN, :]
                     + jnp.dot(hs, Ugv, preferred_element_type=jnp.float32))
            s = jnp.tanh(gates)
            ti = s[:, 0:dg]
            tf = s[:, dg:2 * dg]
            tg = s[:, 2 * dg:3 * dg]
            to = s[:, 3 * dg:4 * dg]
            c = (0.5 + 0.5 * tf) * c + (0.5 + 0.5 * ti) * tg
            hs = (0.5 + 0.5 * to) * jnp.tanh(c)
        out = selfp + jnp.dot(hs, Wn_ref[...],
                              preferred_element_type=jnp.float32)
        out = jnp.where(out > 0, out, _NEG * out)
        if extra is not None:
            W1_ref, b1_ref, W2_ref, b2_ref = rest[:4]
            z = jnp.dot(out, W1_ref[...],
                        preferred_element_type=jnp.float32) + b1_ref[...]
            z = jnp.where(z > 0, z, _NEG * z)
            out = jnp.dot(z, W2_ref[...],
                          preferred_element_type=jnp.float32) + b2_ref[...]
        o_ref[...] = out

    in_specs = [
        pl.BlockSpec((_TN, dp_in), lambda i: (i, 0)),
        pl.BlockSpec((_DEG, _TN, dp_in), lambda i: (0, i, 0)),
        pl.BlockSpec((dp_in, GP), lambda i: (0, 0)),
        pl.BlockSpec((dg, GP), lambda i: (0, 0)),
        pl.BlockSpec((1, GP), lambda i: (0, 0)),
        pl.BlockSpec((dp_in, d_s), lambda i: (0, 0)),
        pl.BlockSpec((1, d_s), lambda i: (0, 0)),
        pl.BlockSpec((dg, d_s), lambda i: (0, 0)),
    ]
    args = [h, neigh, Wg, Ug, bg, Ws, bs, Wn]
    if extra is not None:
        W1, b1, W2, b2 = extra
        in_specs += [
            pl.BlockSpec((dout, 72), lambda i: (0, 0)),
            pl.BlockSpec((1, 72), lambda i: (0, 0)),
            pl.BlockSpec((72, 10), lambda i: (0, 0)),
            pl.BlockSpec((1, 10), lambda i: (0, 0)),
        ]
        args += [W1, b1, W2, b2]

    return pl.pallas_call(
        body,
        grid=(_N // _TN,),
        in_specs=in_specs,
        out_specs=pl.BlockSpec((_TN, d_o), lambda i: (i, 0)),
        out_shape=jax.ShapeDtypeStruct((_N, d_o), jnp.float32),
    )(*args)


def _pack_gates(Wih, Whh, bih, bhh, din, dp_in, dg):
    sc = jnp.array([0.5, 0.5, 1.0, 0.5], jnp.float32)
    Wg = jnp.concatenate(
        [sc[g] * jnp.pad(Wih[g * din:(g + 1) * din, :].T,
                         ((0, dp_in - din), (0, dg - din))) for g in range(4)],
        axis=1)
    Ug = jnp.concatenate(
        [sc[g] * jnp.pad(Whh[g * din:(g + 1) * din, :].T,
                         ((0, dg - din), (0, dg - din))) for g in range(4)],
        axis=1)
    b = bih + bhh
    bg = jnp.concatenate(
        [sc[g] * jnp.pad(b[g * din:(g + 1) * din], (0, dg - din))
         for g in range(4)])[None, :]
    return Wg, Ug, bg


def kernel(x, edge_index, params):
    src = edge_index[0].astype(jnp.int32)
    src_tm = src.reshape(_N, _DEG).T.reshape(-1)

    e0, e1, e2 = params['emb0'], params['emb1'], params['emb2']
    emb_cat = jnp.concatenate([
        jnp.pad(e0, ((0, 0), (0, 6))),
        jnp.pad(e1, ((0, 0), (6, 4))),
        jnp.pad(e2, ((0, 0), (8, 0))),
    ], axis=0)

    h = _embed(x[:, :3], x[:, 3:], emb_cat)

    dims = [(46, 128, 64, 64), (64, 128, 64, 80), (80, 128, 96, 96)]
    for l, (din, dp_in, dg, dout) in enumerate(dims):
        cpad = 0 if l == 2 else 128 - dout
        Wg, Ug, bg = _pack_gates(
            params['l%d_Wih' % l], params['l%d_Whh' % l],
            params['l%d_bih' % l], params['l%d_bhh' % l], din, dp_in, dg)
        Ws = jnp.pad(params['l%d_Wself' % l].T,
                     ((0, dp_in - din), (0, cpad)))
        bs = jnp.pad(params['l%d_bself' % l], (0, cpad))[None, :]
        Wn = jnp.pad(params['l%d_Wneigh' % l].T,
                     ((0, dg - din), (0, cpad)))
        extra = None
        if l == 2:
            extra = (params['lin1_W'].T, params['lin1_b'][None, :],
                     params['lin2_W'].T, params['lin2_b'][None, :])
        neigh = _sc_gather(h, src_tm, dp_in).reshape(_DEG, _N, dp_in)
        h = _sage_layer(h, neigh, Wg, Ug, bg, Ws, bs, Wn, dp_in, dg, dout,
                        extra=extra)
    return h
